# baseline (speedup 1.0000x reference)
import jax
import jax.numpy as jnp
from jax.experimental import pallas as pl
from jax.experimental.pallas import tpu as pltpu

_S = 2048
_IMG_BEGIN, _IMG_END = 128, 1152
_PAD_BEGIN_ENDS = ((0, 80), (0, 100), (0, 110), (0, 0))
_NCH = 4
_CH_ROWS = _S // _NCH


def _mask_kernel(pads_ref, out_ref, scratch, sem):
    b = pl.program_id(0)
    nb = pl.num_programs(0)
    slot = jax.lax.rem(b, 2)
    pe = pads_ref[b, 1]

    def copies(sl, bb):
        return [
            pltpu.make_async_copy(
                scratch.at[sl, pl.ds(i * _CH_ROWS, _CH_ROWS)],
                out_ref.at[bb, pl.ds(i * _CH_ROWS, _CH_ROWS)],
                sem.at[sl, i],
            )
            for i in range(_NCH)
        ]

    @pl.when(b >= 2)
    def _():
        for c in copies(slot, b - 2):
            c.wait()

    q = jax.lax.broadcasted_iota(jnp.int32, (_S, 1), 0)
    in_img = (q >= _IMG_BEGIN) & (q < _IMG_END)
    lo = jnp.minimum(q, pe)
    hi = jnp.where(in_img, _IMG_END, q + 1)
    kv = jax.lax.broadcasted_iota(jnp.int32, (_S, _S), 1)
    scratch[slot] = jnp.where((kv >= lo) & (kv < hi), 1, 0)

    for c in copies(slot, b):
        c.start()

    @pl.when(b == nb - 1)
    def _():
        for c in copies(1 - slot, b - 1):
            c.wait()
        for c in copies(slot, b):
            c.wait()


def kernel(sequence):
    B, S = sequence.shape
    pads = jnp.asarray(_PAD_BEGIN_ENDS, dtype=jnp.int32)
    return pl.pallas_call(
        _mask_kernel,
        grid=(B,),
        in_specs=[pl.BlockSpec(memory_space=pltpu.SMEM)],
        out_specs=pl.BlockSpec(memory_space=pl.ANY),
        out_shape=jax.ShapeDtypeStruct((B, S, S), jnp.bool_),
        scratch_shapes=[
            pltpu.VMEM((2, S, S), jnp.int32),
            pltpu.SemaphoreType.DMA((2, _NCH)),
        ],
    )(pads)
